# Initial kernel scaffold; baseline (speedup 1.0000x reference)
import jax
import jax.numpy as jnp
from jax.experimental import pallas as pl


def kernel(x, Wr, br, W1, b1, W2, b2, Wc, bc):
    raise NotImplementedError("write your pallas kernel here")


# speedup vs baseline: 2.1136x; 2.1136x over previous
import functools

import jax
import jax.numpy as jnp
from jax.experimental import pallas as pl
from jax.experimental.pallas import tpu as pltpu

TOP_K = 2
CAPACITY_FACTOR = 1.25


def _cumsum_excl(v):
    T, E = v.shape
    out = v
    d = 1
    while d < T:
        shifted = jnp.concatenate(
            [jnp.zeros((d, E), v.dtype), out[: T - d, :]], axis=0
        )
        out = out + shifted
        d *= 2
    return out - v


def _routing_weights(xf, Wr, br, T, E, C):
    logits = jnp.dot(xf, Wr, preferred_element_type=jnp.float32) + br
    m = jnp.max(logits, axis=1, keepdims=True)
    p = jnp.exp(logits - m)
    probs = p / jnp.sum(p, axis=1, keepdims=True)
    iota = jax.lax.broadcasted_iota(jnp.int32, (T, E), 1)
    p1 = jnp.max(probs, axis=1, keepdims=True)
    idx1 = jnp.min(jnp.where(probs == p1, iota, E), axis=1, keepdims=True)
    oh1 = iota == idx1
    pm = jnp.where(oh1, -1.0, probs)
    p2 = jnp.max(pm, axis=1, keepdims=True)
    idx2 = jnp.min(jnp.where(pm == p2, iota, E), axis=1, keepdims=True)
    oh2 = iota == idx2
    oh1f = oh1.astype(jnp.float32)
    oh2f = oh2.astype(jnp.float32)
    den = p1 + p2
    g1 = p1 / den
    g2 = p2 / den
    c1 = _cumsum_excl(oh1f)
    tot1 = jnp.sum(oh1f, axis=0, keepdims=True)
    c2 = _cumsum_excl(oh2f)
    pos0 = jnp.sum(c1 * oh1f, axis=1, keepdims=True)
    pos1 = jnp.sum((c2 + tot1) * oh2f, axis=1, keepdims=True)
    v0 = (pos0 < C).astype(jnp.float32)
    v1 = (pos1 < C).astype(jnp.float32)
    return g1 * v0 * oh1f + g2 * v1 * oh2f


def _moe_body(
    xf_ref, Wr_ref, br_ref, W1_ref, b1_ref, W2_ref, b2_ref, Wc_ref, bc_ref,
    out_ref, A_ref, acc_ref, *, T, E, C, S,
):
    e = pl.program_id(0)

    @pl.when(e == 0)
    def _():
        A_ref[...] = _routing_weights(
            xf_ref[...], Wr_ref[...], br_ref[...], T, E, C
        )
        acc_ref[...] = jnp.zeros_like(acc_ref)

    iota = jax.lax.broadcasted_iota(jnp.int32, (T, E), 1)
    a_col = jnp.sum(
        jnp.where(iota == e, A_ref[...], 0.0), axis=1, keepdims=True
    )
    h = jnp.maximum(
        jnp.dot(xf_ref[...], W1_ref[0], preferred_element_type=jnp.float32)
        + b1_ref[0],
        0.0,
    )
    v = jnp.sum(h * a_col, axis=0, keepdims=True)
    s_e = jnp.sum(a_col)
    acc_ref[...] += (
        jnp.dot(v, W2_ref[0], preferred_element_type=jnp.float32)
        + s_e * b2_ref[0]
    )

    @pl.when(e == E - 1)
    def _():
        out_ref[...] = (
            jnp.dot(
                acc_ref[...] / float(S), Wc_ref[...],
                preferred_element_type=jnp.float32,
            )
            + bc_ref[...]
        )


def kernel(x, Wr, br, W1, b1, W2, b2, Wc, bc):
    B, S, D = x.shape
    E, _, F = W1.shape
    NC = Wc.shape[1]
    T = B * S
    C = int(T * TOP_K / E * CAPACITY_FACTOR)
    xf = x.reshape(T, D)

    out = pl.pallas_call(
        functools.partial(_moe_body, T=T, E=E, C=C, S=S),
        grid=(E,),
        in_specs=[
            pl.BlockSpec((T, D), lambda e: (0, 0)),
            pl.BlockSpec((D, E), lambda e: (0, 0)),
            pl.BlockSpec((1, E), lambda e: (0, 0)),
            pl.BlockSpec((1, D, F), lambda e: (e, 0, 0)),
            pl.BlockSpec((1, 1, F), lambda e: (e, 0, 0)),
            pl.BlockSpec((1, F, D), lambda e: (e, 0, 0)),
            pl.BlockSpec((1, 1, D), lambda e: (e, 0, 0)),
            pl.BlockSpec((D, NC), lambda e: (0, 0)),
            pl.BlockSpec((1, NC), lambda e: (0, 0)),
        ],
        out_specs=pl.BlockSpec((1, NC), lambda e: (0, 0)),
        out_shape=jax.ShapeDtypeStruct((1, NC), jnp.float32),
        scratch_shapes=[
            pltpu.VMEM((T, E), jnp.float32),
            pltpu.VMEM((1, D), jnp.float32),
        ],
        compiler_params=pltpu.CompilerParams(
            dimension_semantics=("arbitrary",)
        ),
    )(
        xf,
        Wr,
        br.reshape(1, E),
        W1,
        b1.reshape(E, 1, F),
        W2,
        b2.reshape(E, 1, D),
        Wc,
        bc.reshape(1, NC),
    )
    return out.reshape(B, NC)
